# baseline (speedup 1.0000x reference)
import functools

import jax
import jax.numpy as jnp
from jax import lax
from jax.experimental import pallas as pl
from jax.experimental.pallas import tpu as pltpu
from jax.experimental.pallas import tpu_sc as plsc

N = 10000
E = 320000
FIN = 128
H = 64
C = 10
K = 3
B = 128

NC = 2
NS = 16
NT = NC * NS
CH = 128
NPAD = 10240
EPAD = 327680
NCHW = EPAD // (NT * CH)
NCHWS = EPAD // (NS * CH)
RPT = NPAD // NS
NBUF = 4
LAG = 2
SLAG = 2

TOTCH = EPAD // CH
SLOW_CID = 1
NCH_SLOW = 32
NCH_FAST = (TOTCH - NS * NCH_SLOW) // NS

@functools.cache
def _mesh():
    return plsc.VectorSubcoreMesh(
        core_axis_name="c", subcore_axis_name="s",
        num_cores=NC, num_subcores=NS)


@functools.cache
def _mesh1():
    return plsc.VectorSubcoreMesh(
        core_axis_name="c", subcore_axis_name="s",
        num_cores=1, num_subcores=NS)


@functools.cache
def _sc_degree_call():
    return pl.kernel(
        _sc_degree_body,
        out_type=jax.ShapeDtypeStruct((NT, NPAD), jnp.float32),
        mesh=_mesh(),
        compiler_params=pltpu.CompilerParams(
            needs_layout_passes=False, use_tc_tiling_on_sc=False),
        scratch_types=[
            pltpu.VMEM((NPAD,), jnp.float32),
            pltpu.VMEM((NCHW, CH), jnp.int32),
        ],
    )


def _sc_degree_body(dst_hbm, out_hbm, hist, dbuf):
    cid = lax.axis_index("c")
    sid = lax.axis_index("s")
    wid = sid * NC + cid

    zeros16 = jnp.zeros((16,), jnp.float32)

    def zero_body(i, _):
        hist[pl.ds(i * 16, 16)] = zeros16
        return 0

    lax.fori_loop(0, NPAD // 16, zero_body, 0)

    pltpu.sync_copy(dst_hbm.at[wid], dbuf)

    ones16 = jnp.ones((16,), jnp.float32)

    def chunk_body(c, _):
        for g in range(CH // 16):
            idx = dbuf[c, pl.ds(g * 16, 16)]
            plsc.addupdate_scatter(hist, [idx], ones16)
        return 0

    lax.fori_loop(0, NCHW, chunk_body, 0)
    pltpu.sync_copy(hist, out_hbm.at[wid])


@functools.cache
def _sc_scatter_call():
    return pl.kernel(
        _sc_scatter_body,
        out_type=jax.ShapeDtypeStruct((NC, NPAD, H), jnp.float32),
        mesh=_mesh(),
        compiler_params=pltpu.CompilerParams(
            needs_layout_passes=False, use_tc_tiling_on_sc=False),
        scratch_types=[
            pltpu.VMEM((NCH_FAST, CH), jnp.int32),
            pltpu.VMEM((NCH_FAST, CH), jnp.int32),
            [pltpu.VMEM((CH, H), jnp.float32) for _ in range(NBUF)],
            pltpu.VMEM_SHARED((NPAD, H), jnp.float32),
            [pltpu.SemaphoreType.DMA for _ in range(NBUF)],
            [pltpu.SemaphoreType.DMA for _ in range(NBUF)],
        ],
    )


def _sc_scatter_body(u_hbm, src_hbm, dst_hbm, zrows_hbm, out_hbm,
                     sidx, didx, rows, acc, gsem, ssem):
    cid = lax.axis_index("c")
    sid = lax.axis_index("s")
    ush = u_hbm

    slow = cid == SLOW_CID
    nch = jnp.where(slow, NCH_SLOW, NCH_FAST)
    start = jnp.where(slow, sid * NCH_SLOW, NS * NCH_SLOW + sid * NCH_FAST)

    for z in range(RPT // CH):
        pltpu.sync_copy(zrows_hbm, acc.at[pl.ds(sid * RPT + z * CH, CH)])

    pltpu.sync_copy(src_hbm.at[pl.ds(start, NCH_FAST)], sidx)
    pltpu.sync_copy(dst_hbm.at[pl.ds(start, NCH_FAST)], didx)
    plsc.subcore_barrier()

    def gather(c, b):
        pltpu.async_copy(ush.at[sidx.at[c]], rows[b], gsem[b])

    def gather_wait(c, b):
        pltpu.make_async_copy(ush.at[sidx.at[c]], rows[b], gsem[b]).wait()

    def scatter(c, b):
        pltpu.async_copy(rows[b], acc.at[didx.at[c]], ssem[b], add=True)

    def scatter_wait(c, b):
        pltpu.make_async_copy(rows[b], acc.at[didx.at[c]], ssem[b]).wait()

    for b in range(LAG):
        gather(b, b)
    for b in range(NBUF):
        gather_wait(b, b)
        scatter(b, b)
        if b >= SLAG:
            scatter_wait(b - SLAG, b - SLAG)
        gather(b + LAG, (b + LAG) % NBUF)

    def main_body(t, _):
        c0 = t * NBUF
        for b in range(NBUF):
            c = c0 + b
            gather_wait(c, b)
            scatter(c, b)
            scatter_wait(c - SLAG, (b - SLAG) % NBUF)
            gather(c + LAG, (b + LAG) % NBUF)
        return 0

    lax.fori_loop(1, nch // NBUF - 1, main_body, 0)

    cz = nch - NBUF
    for b in range(NBUF):
        c = cz + b
        gather_wait(c, b)
        scatter(c, b)
        if b < NBUF - LAG:
            scatter_wait(c - SLAG, (b - SLAG) % NBUF)
            gather(c + LAG, (b + LAG) % NBUF)
    for b in range(NBUF):
        scatter_wait(cz + b, b)

    plsc.subcore_barrier()
    pltpu.sync_copy(acc.at[pl.ds(sid * RPT, RPT)],
                    out_hbm.at[cid, pl.ds(sid * RPT, RPT)])


def _tc_prelude_body(degp_ref, xp_ref, w1_ref, u1_ref, dinv_ref):
    ones32 = jnp.ones((NT, 1), jnp.float32)
    deg = lax.dot_general(degp_ref[...], ones32, (((0,), (0,)), ((), ())),
                          preferred_element_type=jnp.float32)
    dinv = lax.rsqrt(deg + 1.0)
    dinv_ref[...] = dinv
    u1_ref[...] = jnp.dot(dinv * xp_ref[...], w1_ref[...],
                          preferred_element_type=jnp.float32)


def _tc_fixup_body(s_ref, u_ref, dinv_ref, b_ref, w_ref, un_ref):
    dinv = dinv_ref[...]
    h = dinv * (s_ref[0] + s_ref[1] + u_ref[...]) + b_ref[...]
    h = jnp.maximum(h, 0.0)
    un_ref[...] = jnp.dot(dinv * h, w_ref[...],
                          preferred_element_type=jnp.float32)


def _tc_head_body(s_ref, u_ref, dinv_ref, b_ref, batch_ref, cent_ref, rb_ref,
                  out_ref):
    h = dinv_ref[...] * (s_ref[0] + s_ref[1] + u_ref[...]) + b_ref[...]
    seg = lax.broadcasted_iota(jnp.int32, (NPAD, B), 1)
    onehot = (batch_ref[...] == seg).astype(jnp.float32)
    dims = (((0,), (0,)), ((), ()))
    sums = lax.dot_general(onehot, h, dims,
                           preferred_element_type=jnp.float32)
    cnt = lax.dot_general(onehot, jnp.ones((NPAD, 1), jnp.float32), dims,
                          preferred_element_type=jnp.float32)
    emb = sums / jnp.maximum(cnt, 1.0)
    esq = jnp.sum(emb * emb, axis=1, keepdims=True)
    embA = jnp.concatenate([-2.0 * emb, jnp.ones((B, 1), jnp.float32)], axis=1)
    onescol = jnp.ones((H, 1), jnp.float32)
    mind = None
    for k in range(K):
        ck = cent_ref[k]
        csq = jnp.dot(ck * ck, onescol,
                      preferred_element_type=jnp.float32)
        cA = jnp.concatenate([ck, csq], axis=1)
        cross = lax.dot_general(embA, cA, (((1,), (1,)), ((), ())),
                                preferred_element_type=jnp.float32)
        d2k = esq + cross
        mind = d2k if mind is None else jnp.minimum(mind, d2k)
    logits = -mind
    rej = rb_ref[...] * jnp.ones((B, 1), jnp.float32)
    out_ref[...] = jnp.concatenate([logits, rej], axis=1)


_tc_prelude = pl.pallas_call(
    _tc_prelude_body,
    out_shape=[jax.ShapeDtypeStruct((NPAD, H), jnp.float32),
               jax.ShapeDtypeStruct((NPAD, 1), jnp.float32)],
)

_tc_fixup = pl.pallas_call(
    _tc_fixup_body,
    out_shape=jax.ShapeDtypeStruct((NPAD, H), jnp.float32),
)

_tc_head = pl.pallas_call(
    _tc_head_body,
    out_shape=jax.ShapeDtypeStruct((B, C + 1), jnp.float32),
)


def kernel(x, edge_index, batch, W1, b1, W2, b2, W3, b3, centroids, reject_bias):
    f32 = jnp.float32
    pad_e = EPAD - E
    src_flat = jnp.concatenate([edge_index[0], jnp.full((pad_e,), N, jnp.int32)])
    dst_flat = jnp.concatenate([edge_index[1], jnp.full((pad_e,), N, jnp.int32)])
    src = src_flat.reshape(TOTCH, CH)
    dst = dst_flat.reshape(TOTCH, CH)
    dst32 = dst_flat.reshape(NT, NCHW, CH)
    xp = jnp.concatenate([x, jnp.zeros((NPAD - N, FIN), f32)], axis=0)
    batch2d = jnp.concatenate(
        [batch, jnp.full((NPAD - N,), B, jnp.int32)]).reshape(NPAD, 1)
    zrows = jnp.zeros((CH, H), f32)
    centK = centroids.reshape(C, K, H).transpose(1, 0, 2)
    rb2d = reject_bias.reshape(1, 1).astype(f32)

    degp = _sc_degree_call()(dst32)
    u1, dinv = _tc_prelude(degp, xp, W1)

    scat = _sc_scatter_call()
    s1 = scat(u1, src, dst, zrows)
    u2 = _tc_fixup(s1, u1, dinv, b1.reshape(1, H), W2)
    s2 = scat(u2, src, dst, zrows)
    u3 = _tc_fixup(s2, u2, dinv, b2.reshape(1, H), W3)
    s3 = scat(u3, src, dst, zrows)

    return _tc_head(s3, u3, dinv, b3.reshape(1, H), batch2d, centK, rb2d)


# speedup vs baseline: 2.8165x; 2.8165x over previous
import functools

import jax
import jax.numpy as jnp
from jax import lax
from jax.experimental import pallas as pl
from jax.experimental.pallas import tpu as pltpu
from jax.experimental.pallas import tpu_sc as plsc

N = 10000
E = 320000
FIN = 128
H = 64
C = 10
K = 3
B = 128

NC = 2
NS = 16
NT = NC * NS
CH = 128
NPAD = 10240
EPAD = 327680
NCHW = EPAD // (NT * CH)
NCHWS = EPAD // (NS * CH)
RPT = NPAD // NS
NBUF = 4
LAG = 2
SLAG = 2

CHS = 64
TOTCH = EPAD // CHS
SLOW_CID = 1
NCH_SLOW = TOTCH // NT
NCH_FAST = (TOTCH - NS * NCH_SLOW) // NS

@functools.cache
def _mesh():
    return plsc.VectorSubcoreMesh(
        core_axis_name="c", subcore_axis_name="s",
        num_cores=NC, num_subcores=NS)


@functools.cache
def _mesh1():
    return plsc.VectorSubcoreMesh(
        core_axis_name="c", subcore_axis_name="s",
        num_cores=1, num_subcores=NS)


@functools.cache
def _sc_degree_call():
    return pl.kernel(
        _sc_degree_body,
        out_type=jax.ShapeDtypeStruct((NT, NPAD), jnp.float32),
        mesh=_mesh(),
        compiler_params=pltpu.CompilerParams(
            needs_layout_passes=False, use_tc_tiling_on_sc=False),
        scratch_types=[
            pltpu.VMEM((NPAD,), jnp.float32),
            pltpu.VMEM((NCHW, CH), jnp.int32),
        ],
    )


def _sc_degree_body(dst_hbm, out_hbm, hist, dbuf):
    cid = lax.axis_index("c")
    sid = lax.axis_index("s")
    wid = sid * NC + cid

    zeros16 = jnp.zeros((16,), jnp.float32)

    def zero_body(i, _):
        hist[pl.ds(i * 16, 16)] = zeros16
        return 0

    lax.fori_loop(0, NPAD // 16, zero_body, 0)

    pltpu.sync_copy(dst_hbm.at[wid], dbuf)

    ones16 = jnp.ones((16,), jnp.float32)

    def chunk_body(c, _):
        for g in range(CH // 16):
            idx = dbuf[c, pl.ds(g * 16, 16)]
            plsc.addupdate_scatter(hist, [idx], ones16)
        return 0

    lax.fori_loop(0, NCHW, chunk_body, 0)
    pltpu.sync_copy(hist, out_hbm.at[wid])


@functools.cache
def _sc_scatter_call():
    return pl.kernel(
        _sc_scatter_body,
        out_type=jax.ShapeDtypeStruct((NC, NPAD, H), jnp.float32),
        mesh=_mesh(),
        compiler_params=pltpu.CompilerParams(
            needs_layout_passes=False, use_tc_tiling_on_sc=False),
        scratch_types=[
            pltpu.VMEM((NCH_FAST, CHS), jnp.int32),
            pltpu.VMEM((NCH_FAST, CHS), jnp.int32),
            [pltpu.VMEM((CHS, H), jnp.float32) for _ in range(NBUF)],
            pltpu.VMEM_SHARED((NPAD, H), jnp.float32),
            pltpu.VMEM_SHARED((NPAD, H), jnp.float32),
            [pltpu.SemaphoreType.DMA for _ in range(NBUF)],
            [pltpu.SemaphoreType.DMA for _ in range(NBUF)],
        ],
    )


def _sc_scatter_body(u_hbm, src_hbm, dst_hbm, zrows_hbm, out_hbm,
                     sidx, didx, rows, acc, ush, gsem, ssem):
    cid = lax.axis_index("c")
    sid = lax.axis_index("s")

    slow = cid == SLOW_CID
    nch = jnp.where(slow, NCH_SLOW, NCH_FAST)
    start = jnp.where(slow, sid * NCH_SLOW, NS * NCH_SLOW + sid * NCH_FAST)

    pltpu.sync_copy(u_hbm.at[pl.ds(sid * RPT, RPT)],
                    ush.at[pl.ds(sid * RPT, RPT)])
    for z in range(RPT // CH):
        pltpu.sync_copy(zrows_hbm, acc.at[pl.ds(sid * RPT + z * CH, CH)])

    pltpu.sync_copy(src_hbm.at[pl.ds(start, NCH_FAST)], sidx)
    pltpu.sync_copy(dst_hbm.at[pl.ds(start, NCH_FAST)], didx)
    plsc.subcore_barrier()

    def gather(c, b):
        pltpu.async_copy(ush.at[sidx.at[c]], rows[b], gsem[b])

    def gather_wait(c, b):
        pltpu.make_async_copy(ush.at[sidx.at[c]], rows[b], gsem[b]).wait()

    def scatter(c, b):
        pltpu.async_copy(rows[b], acc.at[didx.at[c]], ssem[b], add=True)

    def scatter_wait(c, b):
        pltpu.make_async_copy(rows[b], acc.at[didx.at[c]], ssem[b]).wait()

    for b in range(LAG):
        gather(b, b)
    for b in range(NBUF):
        gather_wait(b, b)
        scatter(b, b)
        if b >= SLAG:
            scatter_wait(b - SLAG, b - SLAG)
        gather(b + LAG, (b + LAG) % NBUF)

    def main_body(t, _):
        c0 = t * NBUF
        for b in range(NBUF):
            c = c0 + b
            gather_wait(c, b)
            scatter(c, b)
            scatter_wait(c - SLAG, (b - SLAG) % NBUF)
            gather(c + LAG, (b + LAG) % NBUF)
        return 0

    lax.fori_loop(1, nch // NBUF - 1, main_body, 0)

    cz = nch - NBUF
    for b in range(NBUF):
        c = cz + b
        gather_wait(c, b)
        scatter(c, b)
        if b < NBUF - LAG:
            scatter_wait(c - SLAG, (b - SLAG) % NBUF)
            gather(c + LAG, (b + LAG) % NBUF)
    for b in range(NBUF):
        scatter_wait(cz + b, b)

    plsc.subcore_barrier()
    pltpu.sync_copy(acc.at[pl.ds(sid * RPT, RPT)],
                    out_hbm.at[cid, pl.ds(sid * RPT, RPT)])


def _tc_prelude_body(degp_ref, xp_ref, w1_ref, u1_ref, dinv_ref):
    ones32 = jnp.ones((NT, 1), jnp.float32)
    deg = lax.dot_general(degp_ref[...], ones32, (((0,), (0,)), ((), ())),
                          preferred_element_type=jnp.float32)
    dinv = lax.rsqrt(deg + 1.0)
    dinv_ref[...] = dinv
    u1_ref[...] = jnp.dot(dinv * xp_ref[...], w1_ref[...],
                          preferred_element_type=jnp.float32)


def _tc_fixup_body(s_ref, u_ref, dinv_ref, b_ref, w_ref, un_ref):
    dinv = dinv_ref[...]
    h = dinv * (s_ref[0] + s_ref[1] + u_ref[...]) + b_ref[...]
    h = jnp.maximum(h, 0.0)
    un_ref[...] = jnp.dot(dinv * h, w_ref[...],
                          preferred_element_type=jnp.float32)


def _tc_head_body(s_ref, u_ref, dinv_ref, b_ref, batch_ref, cent_ref, rb_ref,
                  out_ref):
    h = dinv_ref[...] * (s_ref[0] + s_ref[1] + u_ref[...]) + b_ref[...]
    seg = lax.broadcasted_iota(jnp.int32, (NPAD, B), 1)
    onehot = (batch_ref[...] == seg).astype(jnp.float32)
    dims = (((0,), (0,)), ((), ()))
    sums = lax.dot_general(onehot, h, dims,
                           preferred_element_type=jnp.float32)
    cnt = lax.dot_general(onehot, jnp.ones((NPAD, 1), jnp.float32), dims,
                          preferred_element_type=jnp.float32)
    emb = sums / jnp.maximum(cnt, 1.0)
    esq = jnp.sum(emb * emb, axis=1, keepdims=True)
    embA = jnp.concatenate([-2.0 * emb, jnp.ones((B, 1), jnp.float32)], axis=1)
    onescol = jnp.ones((H, 1), jnp.float32)
    mind = None
    for k in range(K):
        ck = cent_ref[k]
        csq = jnp.dot(ck * ck, onescol,
                      preferred_element_type=jnp.float32)
        cA = jnp.concatenate([ck, csq], axis=1)
        cross = lax.dot_general(embA, cA, (((1,), (1,)), ((), ())),
                                preferred_element_type=jnp.float32)
        d2k = esq + cross
        mind = d2k if mind is None else jnp.minimum(mind, d2k)
    logits = -mind
    rej = rb_ref[...] * jnp.ones((B, 1), jnp.float32)
    out_ref[...] = jnp.concatenate([logits, rej], axis=1)


_tc_prelude = pl.pallas_call(
    _tc_prelude_body,
    out_shape=[jax.ShapeDtypeStruct((NPAD, H), jnp.float32),
               jax.ShapeDtypeStruct((NPAD, 1), jnp.float32)],
)

_tc_fixup = pl.pallas_call(
    _tc_fixup_body,
    out_shape=jax.ShapeDtypeStruct((NPAD, H), jnp.float32),
)

_tc_head = pl.pallas_call(
    _tc_head_body,
    out_shape=jax.ShapeDtypeStruct((B, C + 1), jnp.float32),
)


def kernel(x, edge_index, batch, W1, b1, W2, b2, W3, b3, centroids, reject_bias):
    f32 = jnp.float32
    pad_e = EPAD - E
    src_flat = jnp.concatenate([edge_index[0], jnp.full((pad_e,), N, jnp.int32)])
    dst_flat = jnp.concatenate([edge_index[1], jnp.full((pad_e,), N, jnp.int32)])
    src = src_flat.reshape(TOTCH, CHS)
    dst = dst_flat.reshape(TOTCH, CHS)
    dst32 = dst_flat.reshape(NT, NCHW, CH)
    xp = jnp.concatenate([x, jnp.zeros((NPAD - N, FIN), f32)], axis=0)
    batch2d = jnp.concatenate(
        [batch, jnp.full((NPAD - N,), B, jnp.int32)]).reshape(NPAD, 1)
    zrows = jnp.zeros((CH, H), f32)
    centK = centroids.reshape(C, K, H).transpose(1, 0, 2)
    rb2d = reject_bias.reshape(1, 1).astype(f32)

    degp = _sc_degree_call()(dst32)
    u1, dinv = _tc_prelude(degp, xp, W1)

    scat = _sc_scatter_call()
    s1 = scat(u1, src, dst, zrows)
    u2 = _tc_fixup(s1, u1, dinv, b1.reshape(1, H), W2)
    s2 = scat(u2, src, dst, zrows)
    u3 = _tc_fixup(s2, u2, dinv, b2.reshape(1, H), W3)
    s3 = scat(u3, src, dst, zrows)

    return _tc_head(s3, u3, dinv, b3.reshape(1, H), batch2d, centK, rb2d)
